# baseline (speedup 1.0000x reference)
import functools

import jax
import jax.numpy as jnp
from jax import lax
from jax.experimental import pallas as pl
from jax.experimental.pallas import tpu as pltpu
from jax.experimental.pallas import tpu_sc as plsc

NC = 2
NS = 16
NW = NC * NS
NBUF = 8
LAG = 4


@functools.lru_cache(maxsize=None)
def _build(b: int, s: int, d: int):
  assert b % (NW * NBUF) == 0
  rows_per_w = b // NW
  nchunk = rows_per_w

  mesh = plsc.VectorSubcoreMesh(
      core_axis_name="c", subcore_axis_name="s", num_cores=NC, num_subcores=NS
  )

  @functools.partial(
      pl.kernel,
      mesh=mesh,
      out_type=jax.ShapeDtypeStruct((b, s, d), jnp.float32),
      scratch_types=[
          pltpu.VMEM((NBUF, s), jnp.int32),
          pltpu.VMEM((NBUF, s, d), jnp.float32),
          [pltpu.SemaphoreType.DMA] * NBUF,
          [pltpu.SemaphoreType.DMA] * NBUF,
      ],
      compiler_params=pltpu.CompilerParams(use_tc_tiling_on_sc=False),
  )
  def gather(tok_hbm, table_hbm, out_hbm, idx_v, rows_v, gsem, osem):
    wid = lax.axis_index("s") * NC + lax.axis_index("c")
    base = wid * rows_per_w

    def front(g, bf, wait_out):
      if wait_out:
        pltpu.make_async_copy(rows_v.at[bf], out_hbm.at[0], osem[bf]).wait()
      pltpu.sync_copy(tok_hbm.at[base + g], idx_v.at[bf])
      pltpu.async_copy(table_hbm.at[idx_v.at[bf]], rows_v.at[bf], gsem[bf])

    def back(g, bf):
      pltpu.make_async_copy(
          table_hbm.at[idx_v.at[bf]], rows_v.at[bf], gsem[bf]
      ).wait()
      pltpu.async_copy(rows_v.at[bf], out_hbm.at[base + g], osem[bf])

    for g in range(NBUF):
      front(g, g, wait_out=False)
    for g in range(NBUF - LAG):
      back(g, g)

    @pl.loop(1, nchunk // NBUF)
    def _blk(i):
      for bf in range(NBUF):
        g = i * NBUF + bf
        back(g - LAG, (bf - LAG) % NBUF)
        front(g, bf, wait_out=True)

    for k in range(LAG):
      g = nchunk - LAG + k
      back(g, g % NBUF)
    for bf in range(NBUF):
      pltpu.make_async_copy(rows_v.at[bf], out_hbm.at[0], osem[bf]).wait()

  return gather


def kernel(token, embeddings):
  b, s = token.shape
  d = embeddings.shape[1]
  return _build(b, s, d)(token.astype(jnp.int32), embeddings)
